# Initial kernel scaffold; baseline (speedup 1.0000x reference)
import jax
import jax.numpy as jnp
from jax.experimental import pallas as pl


def kernel(x, W1, b1, W2, b2):
    raise NotImplementedError("write your pallas kernel here")


# speedup vs baseline: 6.7028x; 6.7028x over previous
import functools

import jax
import jax.numpy as jnp
from jax.experimental import pallas as pl
from jax.experimental.pallas import tpu as pltpu

N = 3136
NP = 3200
K = 16
KP1 = 17
RB = 640
_NEG = -1.0


def _knn_body(xb_ref, xa_ref, out_ref, dist_ref):
    xb = xb_ref[...]
    xa = xa_ref[...]
    sqb = jnp.sum(xb * xb, axis=1)
    sqa = jnp.sum(xa * xa, axis=1)
    dot = jax.lax.dot_general(xb, xa, (((1,), (1,)), ((), ())),
                              preferred_element_type=jnp.float32)
    d2 = sqb[:, None] + sqa[None, :] - 2.0 * dot
    d2 = jnp.maximum(d2, 0.0)
    dist = jnp.sqrt(d2)
    colid = jax.lax.broadcasted_iota(jnp.int32, (RB, NP), 1)
    dist_ref[...] = jnp.where(colid < N, dist, jnp.inf)

    r0 = pl.program_id(0) * RB
    rowid = r0 + jax.lax.broadcasted_iota(jnp.int32, (RB,), 0)
    row_ok = rowid < N

    last_v = jnp.full((RB,), _NEG, jnp.float32)
    last_i = jnp.full((RB,), -1, jnp.int32)
    big_i = jnp.int32(NP)
    for t in range(KP1):
        d = dist_ref[...]
        gt = (d > last_v[:, None]) | ((d == last_v[:, None])
                                      & (colid > last_i[:, None]))
        v = jnp.min(jnp.where(gt, d, jnp.inf), axis=1)
        a = jnp.min(jnp.where(gt & (d == v[:, None]), colid, big_i), axis=1)
        out_ref[t, :] = jnp.where(row_ok, a, big_i)
        last_v, last_i = v, a


def _knn(x_pad):
    grid = NP // RB
    return pl.pallas_call(
        _knn_body,
        grid=(grid,),
        in_specs=[
            pl.BlockSpec((RB, 256), lambda i: (i, 0)),
            pl.BlockSpec((NP, 256), lambda i: (0, 0)),
        ],
        out_specs=pl.BlockSpec((KP1, RB), lambda i: (0, i)),
        out_shape=jax.ShapeDtypeStruct((KP1, NP), jnp.int32),
        scratch_shapes=[pltpu.VMEM((RB, NP), jnp.float32)],
    )(x_pad, x_pad)


def _deg_body(cols_ref, dinv_ref):
    d0 = pl.program_id(0) * 128
    did = d0 + jax.lax.broadcasted_iota(jnp.int32, (128,), 0)
    cols = cols_ref[...]
    cnt = jnp.sum(
        (cols[None, :, :] == did[:, None, None]).astype(jnp.float32),
        axis=(1, 2))
    deg = cnt + 1.0
    dinv_ref[...] = 1.0 / jnp.sqrt(deg)


def _deg(cols):
    return pl.pallas_call(
        _deg_body,
        grid=(NP // 128,),
        in_specs=[pl.BlockSpec((K, NP), lambda i: (0, 0))],
        out_specs=pl.BlockSpec((128,), lambda i: (i,)),
        out_shape=jax.ShapeDtypeStruct((NP,), jnp.float32),
    )(cols)


def _mm_scale_body(h_ref, w_ref, dinv_ref, y_ref):
    r0 = pl.program_id(0) * RB
    y = jax.lax.dot_general(h_ref[...], w_ref[...], (((1,), (0,)), ((), ())),
                            preferred_element_type=jnp.float32)
    y_ref[...] = dinv_ref[pl.ds(r0, RB)][:, None] * y


def _mm_scale(h, w, dinv):
    grid = NP // RB
    return pl.pallas_call(
        _mm_scale_body,
        grid=(grid,),
        in_specs=[
            pl.BlockSpec((RB, 256), lambda i: (i, 0)),
            pl.BlockSpec((256, 256), lambda i: (0, 0)),
            pl.BlockSpec((NP,), lambda i: (0,)),
        ],
        out_specs=pl.BlockSpec((RB, 256), lambda i: (i, 0)),
        out_shape=jax.ShapeDtypeStruct((NP, 256), jnp.float32),
    )(h, w, dinv)


def _layer_body(cols_ref, y_ref, dinv_ref, b_ref, out_ref):
    r0 = pl.program_id(0) * RB
    rowid = r0 + jax.lax.broadcasted_iota(jnp.int32, (RB, 1), 0)
    cols = cols_ref[...]
    a_blk = jnp.zeros((RB, NP), jnp.float32)
    for k in range(K):
        a_blk = a_blk + (cols[k, :][None, :] == rowid).astype(jnp.float32)
    acc = jax.lax.dot_general(a_blk, y_ref[...], (((1,), (0,)), ((), ())),
                              preferred_element_type=jnp.float32)
    acc = acc + y_ref[pl.ds(r0, RB), :]
    h = jax.nn.relu(dinv_ref[pl.ds(r0, RB)][:, None] * acc + b_ref[...][None, :])
    out_ref[...] = jnp.where(rowid < N, h, 0.0)


def _layer(cols, y, dinv, b):
    grid = NP // RB
    return pl.pallas_call(
        _layer_body,
        grid=(grid,),
        in_specs=[
            pl.BlockSpec((K, NP), lambda i: (0, 0)),
            pl.BlockSpec((NP, 256), lambda i: (0, 0)),
            pl.BlockSpec((NP,), lambda i: (0,)),
            pl.BlockSpec((256,), lambda i: (0,)),
        ],
        out_specs=pl.BlockSpec((RB, 256), lambda i: (i, 0)),
        out_shape=jax.ShapeDtypeStruct((NP, 256), jnp.float32),
    )(cols, y, dinv, b)


@jax.jit
def kernel(x, W1, b1, W2, b2):
    B, H, Wd, C = x.shape
    x_flat = x.reshape(B * H * Wd, C)
    x_pad = jnp.zeros((NP, C), x_flat.dtype).at[:N].set(x_flat)

    knn = _knn(x_pad)
    cols = knn[1:]

    dinv = _deg(cols)
    y1 = _mm_scale(x_pad, W1, dinv)
    h1 = _layer(cols, y1, dinv, b1)
    y2 = _mm_scale(h1, W2, dinv)
    h2 = _layer(cols, y2, dinv, b2)
    return h2[:N].reshape(B, H, Wd, -1)
